# Initial kernel scaffold; baseline (speedup 1.0000x reference)
import functools

import jax
import jax.numpy as jnp
from jax import lax
from jax.experimental import pallas as pl
from jax.experimental.pallas import tpu as pltpu
from jax.experimental.pallas import tpu_sc as plsc

N = 10000
D = 128
E = 320000
NC = 2
NS = 16
NW = NC * NS
EPW = E // NW
KB = 125
NBLK = EPW // KB
RPT = N // NS
CHUNK = 125
NCHUNK = RPT // CHUNK
LANES = 16
RB = 1000
GRID = N // RB


def _pre_body(x_ref, w1t_ref, w2t_ref, b_ref, a_ref, bp_ref):
    xb = x_ref[...]
    a_ref[...] = jnp.dot(xb, w1t_ref[...], preferred_element_type=jnp.float32)
    bp_ref[...] = (
        jnp.dot(xb, w2t_ref[...], preferred_element_type=jnp.float32)
        + b_ref[...]
    )


_pre_call = pl.pallas_call(
    _pre_body,
    grid=(GRID,),
    in_specs=[
        pl.BlockSpec((RB, D), lambda i: (i, 0)),
        pl.BlockSpec((D, D), lambda i: (0, 0)),
        pl.BlockSpec((D, D), lambda i: (0, 0)),
        pl.BlockSpec((1, D), lambda i: (0, 0)),
    ],
    out_specs=[
        pl.BlockSpec((RB, D), lambda i: (i, 0)),
        pl.BlockSpec((RB, D), lambda i: (i, 0)),
    ],
    out_shape=[
        jax.ShapeDtypeStruct((N, D), jnp.float32),
        jax.ShapeDtypeStruct((N, D), jnp.float32),
    ],
)


def _post_body(p_ref, degp_ref, bp_ref, x_ref, o_ref):
    d = degp_ref[0] + degp_ref[1]
    dcol = d[:, 0:1]
    o_ref[...] = p_ref[0] + p_ref[1] + x_ref[...] + dcol * bp_ref[...]


_post_call = pl.pallas_call(
    _post_body,
    grid=(GRID,),
    in_specs=[
        pl.BlockSpec((NC, RB, D), lambda i: (0, i, 0)),
        pl.BlockSpec((NC, RB, LANES), lambda i: (0, i, 0)),
        pl.BlockSpec((RB, D), lambda i: (i, 0)),
        pl.BlockSpec((RB, D), lambda i: (i, 0)),
    ],
    out_specs=pl.BlockSpec((RB, D), lambda i: (i, 0)),
    out_shape=jax.ShapeDtypeStruct((N, D), jnp.float32),
)


_MESH = plsc.VectorSubcoreMesh(
    core_axis_name="c", subcore_axis_name="s", num_cores=NC, num_subcores=NS
)


def _fill_rows(ref, nrows, ncols, val):
    vec = jnp.full((LANES,), val, jnp.float32)

    def body(i, carry):
        for j in range(ncols // LANES):
            ref[i, pl.ds(j * LANES, LANES)] = vec
        return carry

    lax.fori_loop(0, nrows, body, 0)


@functools.partial(
    pl.kernel,
    out_type=[
        jax.ShapeDtypeStruct((NC, N, D), jnp.float32),
        jax.ShapeDtypeStruct((NC, N, LANES), jnp.float32),
    ],
    mesh=_MESH,
    scratch_types=[
        pltpu.VMEM((NBLK, KB), jnp.int32),
        pltpu.VMEM((NBLK, KB), jnp.int32),
        pltpu.VMEM((KB, D), jnp.float32),
        pltpu.VMEM((KB, D), jnp.float32),
        pltpu.VMEM((KB, LANES), jnp.float32),
        pltpu.VMEM((RPT, LANES), jnp.float32),
        pltpu.VMEM_SHARED((N, D), jnp.float32),
        pltpu.VMEM_SHARED((N, LANES), jnp.float32),
        pltpu.SemaphoreType.DMA,
        pltpu.SemaphoreType.DMA,
    ],
)
def _sc_scatter(a_hbm, row_hbm, col_hbm, out_hbm, deg_hbm,
                row_v, col_v, g0, g1, ones_v, dbuf, acc, dacc, sem0, sem1):
    cid = lax.axis_index("c")
    sid = lax.axis_index("s")
    wid = sid * NC + cid

    pltpu.sync_copy(row_hbm.at[wid], row_v)
    pltpu.sync_copy(col_hbm.at[wid], col_v)

    _fill_rows(g0, CHUNK, D, 0.0)
    for ch in range(NCHUNK):
        base = sid * RPT + ch * CHUNK
        pltpu.sync_copy(g0, acc.at[pl.ds(base, CHUNK)])
    _fill_rows(dbuf, RPT, LANES, 0.0)
    pltpu.sync_copy(dbuf, dacc.at[pl.ds(sid * RPT, RPT)])
    _fill_rows(ones_v, KB, LANES, 1.0)
    plsc.subcore_barrier()

    def body(i, carry):
        h0 = pltpu.async_copy(a_hbm.at[row_v.at[2 * i]], g0, sem0)
        h1 = pltpu.async_copy(a_hbm.at[row_v.at[2 * i + 1]], g1, sem1)
        h0.wait()
        pltpu.sync_copy(g0, acc.at[col_v.at[2 * i]], add=True)
        pltpu.sync_copy(ones_v, dacc.at[col_v.at[2 * i]], add=True)
        h1.wait()
        pltpu.sync_copy(g1, acc.at[col_v.at[2 * i + 1]], add=True)
        pltpu.sync_copy(ones_v, dacc.at[col_v.at[2 * i + 1]], add=True)
        return carry

    lax.fori_loop(0, NBLK // 2, body, 0)
    plsc.subcore_barrier()

    for ch in range(NCHUNK):
        base = sid * RPT + ch * CHUNK
        pltpu.sync_copy(acc.at[pl.ds(base, CHUNK)], g0)
        pltpu.sync_copy(g0, out_hbm.at[cid].at[pl.ds(base, CHUNK)])
    pltpu.sync_copy(dacc.at[pl.ds(sid * RPT, RPT)], dbuf)
    pltpu.sync_copy(dbuf, deg_hbm.at[cid].at[pl.ds(sid * RPT, RPT)])


def kernel(x, edge_index, W_msg, b_msg):
    w1t = W_msg[:, :D].T
    w2t = W_msg[:, D:].T
    a, bp = _pre_call(x, w1t, w2t, b_msg.reshape(1, D))
    row3 = edge_index[0].reshape(NW, NBLK, KB)
    col3 = edge_index[1].reshape(NW, NBLK, KB)
    partials, degp = _sc_scatter(a, row3, col3)
    return _post_call(partials, degp, bp, x)


# speedup vs baseline: 5.4761x; 5.4761x over previous
import functools

import jax
import jax.numpy as jnp
from jax import lax
from jax.experimental import pallas as pl
from jax.experimental.pallas import tpu as pltpu
from jax.experimental.pallas import tpu_sc as plsc

N = 10000
NPAD = 10240
D = 128
E = 320000
NC = 2
NS = 16
NW = NC * NS
EPW = E // NW
KB = 128
NBLK = 80
EPWP = NBLK * KB
RPT = NPAD // NS
CHUNK = 128
NCHUNK = RPT // CHUNK
LANES = 16
RB = 1000
GRID = N // RB


def _pre_body(x_ref, w1t_ref, w2t_ref, b_ref, a_ref, bp_ref):
    xb = x_ref[...]
    a_ref[...] = jnp.dot(xb, w1t_ref[...], preferred_element_type=jnp.float32)
    bp_ref[...] = (
        jnp.dot(xb, w2t_ref[...], preferred_element_type=jnp.float32)
        + b_ref[...]
    )


_pre_call = pl.pallas_call(
    _pre_body,
    grid=(GRID,),
    in_specs=[
        pl.BlockSpec((RB, D), lambda i: (i, 0)),
        pl.BlockSpec((D, D), lambda i: (0, 0)),
        pl.BlockSpec((D, D), lambda i: (0, 0)),
        pl.BlockSpec((1, D), lambda i: (0, 0)),
    ],
    out_specs=[
        pl.BlockSpec((RB, D), lambda i: (i, 0)),
        pl.BlockSpec((RB, D), lambda i: (i, 0)),
    ],
    out_shape=[
        jax.ShapeDtypeStruct((N, D), jnp.float32),
        jax.ShapeDtypeStruct((N, D), jnp.float32),
    ],
)


def _post_body(p_ref, degp_ref, bp_ref, x_ref, o_ref):
    d = degp_ref[0] + degp_ref[1]
    dcol = d[:, 0:1]
    o_ref[...] = p_ref[0] + p_ref[1] + x_ref[...] + dcol * bp_ref[...]


_post_call = pl.pallas_call(
    _post_body,
    grid=(GRID,),
    in_specs=[
        pl.BlockSpec((NC, RB, D), lambda i: (0, i, 0)),
        pl.BlockSpec((NC, RB, LANES), lambda i: (0, i, 0)),
        pl.BlockSpec((RB, D), lambda i: (i, 0)),
        pl.BlockSpec((RB, D), lambda i: (i, 0)),
    ],
    out_specs=pl.BlockSpec((RB, D), lambda i: (i, 0)),
    out_shape=jax.ShapeDtypeStruct((N, D), jnp.float32),
)


_MESH = plsc.VectorSubcoreMesh(
    core_axis_name="c", subcore_axis_name="s", num_cores=NC, num_subcores=NS
)


def _fill_rows(ref, nrows, ncols, val):
    vec = jnp.full((LANES,), val, jnp.float32)

    def body(i, carry):
        for j in range(ncols // LANES):
            ref[i, pl.ds(j * LANES, LANES)] = vec
        return carry

    lax.fori_loop(0, nrows, body, 0)


@functools.partial(
    pl.kernel,
    out_type=[
        jax.ShapeDtypeStruct((NC, NPAD, D), jnp.float32),
        jax.ShapeDtypeStruct((NC, NPAD, LANES), jnp.float32),
    ],
    mesh=_MESH,
    compiler_params=pltpu.CompilerParams(use_tc_tiling_on_sc=False),
    scratch_types=[
        pltpu.VMEM((KB,), jnp.int32),
        pltpu.VMEM((KB,), jnp.int32),
        pltpu.VMEM((KB,), jnp.int32),
        pltpu.VMEM((KB,), jnp.int32),
        pltpu.VMEM((KB, D), jnp.float32),
        pltpu.VMEM((KB, D), jnp.float32),
        pltpu.VMEM((KB, LANES), jnp.float32),
        pltpu.VMEM_SHARED((NPAD, D), jnp.float32),
        pltpu.VMEM_SHARED((NPAD, LANES), jnp.float32),
        pltpu.SemaphoreType.DMA,
        pltpu.SemaphoreType.DMA,
    ],
)
def _sc_scatter(a_hbm, row_hbm, col_hbm, out_hbm, deg_hbm,
                ri0, ri1, ci0, ci1, g0, g1, ones_v, acc, dacc,
                sem0, sem1):
    cid = lax.axis_index("c")
    sid = lax.axis_index("s")
    wid = sid * NC + cid

    _fill_rows(g0, KB, D, 0.0)
    _fill_rows(ones_v, KB, LANES, 0.0)
    for ch in range(NCHUNK):
        base = sid * RPT + ch * CHUNK
        pltpu.sync_copy(g0, acc.at[pl.ds(base, CHUNK)])
        pltpu.sync_copy(ones_v, dacc.at[pl.ds(base, CHUNK), :])
    _fill_rows(ones_v, KB, LANES, 1.0)
    plsc.subcore_barrier()

    pltpu.sync_copy(row_hbm.at[wid, 0], ri0)
    pltpu.sync_copy(col_hbm.at[wid, 0], ci0)

    def body(i, carry):
        h0 = pltpu.async_copy(a_hbm.at[ri0], g0, sem0)
        pltpu.sync_copy(row_hbm.at[wid, 2 * i + 1], ri1)
        pltpu.sync_copy(col_hbm.at[wid, 2 * i + 1], ci1)
        h1 = pltpu.async_copy(a_hbm.at[ri1], g1, sem1)
        h0.wait()
        pltpu.sync_copy(g0, acc.at[ci0], add=True)
        pltpu.sync_copy(ones_v, dacc.at[ci0], add=True)

        @pl.when(i < NBLK // 2 - 1)
        def _prefetch():
            pltpu.sync_copy(row_hbm.at[wid, 2 * i + 2], ri0)
            pltpu.sync_copy(col_hbm.at[wid, 2 * i + 2], ci0)

        h1.wait()
        pltpu.sync_copy(g1, acc.at[ci1], add=True)
        pltpu.sync_copy(ones_v, dacc.at[ci1], add=True)
        return carry

    lax.fori_loop(0, NBLK // 2, body, 0)
    plsc.subcore_barrier()

    base = sid * RPT
    pltpu.sync_copy(acc.at[pl.ds(base, RPT)], out_hbm.at[cid, pl.ds(base, RPT)])
    pltpu.sync_copy(dacc.at[pl.ds(base, RPT)], deg_hbm.at[cid, pl.ds(base, RPT)])


def kernel(x, edge_index, W_msg, b_msg):
    w1t = W_msg[:, :D].T
    w2t = W_msg[:, D:].T
    a, bp = _pre_call(x, w1t, w2t, b_msg.reshape(1, D))
    pad = EPWP - EPW
    row3 = jnp.pad(edge_index[0].reshape(NW, EPW), ((0, 0), (0, pad)),
                   constant_values=0).reshape(NW, NBLK, KB)
    col3 = jnp.pad(edge_index[1].reshape(NW, EPW), ((0, 0), (0, pad)),
                   constant_values=N).reshape(NW, NBLK, KB)
    partials, degp = _sc_scatter(a, row3, col3)
    return _post_call(partials, degp, bp, x)


# speedup vs baseline: 9.4030x; 1.7171x over previous
import functools

import jax
import jax.numpy as jnp
from jax import lax
from jax.experimental import pallas as pl
from jax.experimental.pallas import tpu as pltpu
from jax.experimental.pallas import tpu_sc as plsc

N = 10000
NPAD = 10240
D = 128
E = 320000
NC = 2
NS = 16
NW = NC * NS
EPW = E // NW
KB = 128
NBLK = 80
EPWP = NBLK * KB
RPT = NPAD // NS
CHUNK = 128
NCHUNK = RPT // CHUNK
LANES = 16
RB = 1000
GRID = N // RB


def _pre_body(x_ref, w1t_ref, w2t_ref, b_ref, a_ref, bp_ref):
    xb = x_ref[...]
    a_ref[...] = jnp.dot(
        xb, w1t_ref[...], preferred_element_type=jnp.float32
    ).astype(jnp.bfloat16)
    bp_ref[...] = (
        jnp.dot(xb, w2t_ref[...], preferred_element_type=jnp.float32)
        + b_ref[...]
    )


_pre_call = pl.pallas_call(
    _pre_body,
    grid=(GRID,),
    in_specs=[
        pl.BlockSpec((RB, D), lambda i: (i, 0)),
        pl.BlockSpec((D, D), lambda i: (0, 0)),
        pl.BlockSpec((D, D), lambda i: (0, 0)),
        pl.BlockSpec((1, D), lambda i: (0, 0)),
    ],
    out_specs=[
        pl.BlockSpec((RB, D), lambda i: (i, 0)),
        pl.BlockSpec((RB, D), lambda i: (i, 0)),
    ],
    out_shape=[
        jax.ShapeDtypeStruct((N, D), jnp.bfloat16),
        jax.ShapeDtypeStruct((N, D), jnp.float32),
    ],
)


def _post_body(p_ref, degp_ref, bp_ref, x_ref, o_ref):
    d = degp_ref[0] + degp_ref[1]
    dcol = d[:, 0:1]
    p = p_ref[0].astype(jnp.float32) + p_ref[1].astype(jnp.float32)
    o_ref[...] = p + x_ref[...] + dcol * bp_ref[...]


_post_call = pl.pallas_call(
    _post_body,
    grid=(GRID,),
    in_specs=[
        pl.BlockSpec((NC, RB, D), lambda i: (0, i, 0)),
        pl.BlockSpec((NC, RB, LANES), lambda i: (0, i, 0)),
        pl.BlockSpec((RB, D), lambda i: (i, 0)),
        pl.BlockSpec((RB, D), lambda i: (i, 0)),
    ],
    out_specs=pl.BlockSpec((RB, D), lambda i: (i, 0)),
    out_shape=jax.ShapeDtypeStruct((N, D), jnp.float32),
)


_MESH = plsc.VectorSubcoreMesh(
    core_axis_name="c", subcore_axis_name="s", num_cores=NC, num_subcores=NS
)


def _fill_rows(ref, nrows, ncols, val, dtype):
    lanes = 32 if dtype == jnp.bfloat16 else LANES
    vec = jnp.full((lanes,), val, dtype)

    def body(i, carry):
        for j in range(ncols // lanes):
            ref[i, pl.ds(j * lanes, lanes)] = vec
        return carry

    lax.fori_loop(0, nrows, body, 0)


@functools.partial(
    pl.kernel,
    out_type=[
        jax.ShapeDtypeStruct((NC, NPAD, D), jnp.bfloat16),
        jax.ShapeDtypeStruct((NC, NPAD, LANES), jnp.float32),
    ],
    mesh=_MESH,
    compiler_params=pltpu.CompilerParams(use_tc_tiling_on_sc=False),
    scratch_types=[
        pltpu.VMEM((NBLK, KB), jnp.int32),
        pltpu.VMEM((NBLK, KB), jnp.int32),
        pltpu.VMEM((KB, D), jnp.bfloat16),
        pltpu.VMEM((KB, D), jnp.bfloat16),
        pltpu.VMEM((KB, D), jnp.bfloat16),
        pltpu.VMEM((KB, D), jnp.bfloat16),
        pltpu.VMEM((KB, LANES), jnp.float32),
        pltpu.VMEM_SHARED((NPAD, D), jnp.bfloat16),
        pltpu.VMEM_SHARED((NPAD, LANES), jnp.float32),
        [pltpu.SemaphoreType.DMA] * 4,
        [pltpu.SemaphoreType.DMA] * 4,
        [pltpu.SemaphoreType.DMA] * 4,
    ],
)
def _sc_scatter(a_hbm, row_hbm, col_hbm, out_hbm, deg_hbm,
                row_v, col_v, g0, g1, g2, g3, ones_v, acc, dacc,
                gsems, ssems, osems):
    cid = lax.axis_index("c")
    sid = lax.axis_index("s")
    wid = sid * NC + cid
    gs = [g0, g1, g2, g3]

    pltpu.sync_copy(row_hbm.at[wid], row_v)
    pltpu.sync_copy(col_hbm.at[wid], col_v)

    _fill_rows(g0, KB, D, 0.0, jnp.bfloat16)
    _fill_rows(ones_v, KB, LANES, 0.0, jnp.float32)
    for ch in range(NCHUNK):
        base = sid * RPT + ch * CHUNK
        pltpu.sync_copy(g0, acc.at[pl.ds(base, CHUNK)])
        pltpu.sync_copy(ones_v, dacc.at[pl.ds(base, CHUNK), :])
    _fill_rows(ones_v, KB, LANES, 1.0, jnp.float32)
    plsc.subcore_barrier()

    def wait_gather(b):
        pltpu.make_async_copy(a_hbm.at[pl.ds(0, KB)], gs[b], gsems[b]).wait()

    def wait_scatter(b):
        pltpu.make_async_copy(a_hbm.at[pl.ds(0, KB)], gs[b], ssems[b]).wait()

    def wait_ones(b):
        pltpu.make_async_copy(
            deg_hbm.at[cid, pl.ds(0, KB)], ones_v, osems[b]
        ).wait()

    def start_gather(b, j):
        pltpu.async_copy(a_hbm.at[row_v.at[j]], gs[b], gsems[b])

    def visit(b, j, first):
        wait_gather(b)

        @pl.when(jnp.logical_not(first))
        def _():
            wait_ones(b)

        pltpu.async_copy(gs[b], acc.at[col_v.at[j]], ssems[b], add=True)
        pltpu.async_copy(ones_v, dacc.at[col_v.at[j]], osems[b], add=True)

    start_gather(0, 0)
    start_gather(1, 1)

    def body(i, carry):
        first = i == 0
        start_gather(2, 4 * i + 2)
        start_gather(3, 4 * i + 3)
        visit(0, 4 * i, first)
        visit(1, 4 * i + 1, first)
        wait_scatter(0)
        wait_scatter(1)

        @pl.when(i < NBLK // 4 - 1)
        def _():
            start_gather(0, 4 * i + 4)
            start_gather(1, 4 * i + 5)

        visit(2, 4 * i + 2, first)
        visit(3, 4 * i + 3, first)
        wait_scatter(2)
        wait_scatter(3)
        return carry

    lax.fori_loop(0, NBLK // 4, body, 0)
    for b in range(4):
        wait_ones(b)
    plsc.subcore_barrier()

    base = sid * RPT
    pltpu.sync_copy(acc.at[pl.ds(base, RPT)], out_hbm.at[cid, pl.ds(base, RPT)])
    pltpu.sync_copy(dacc.at[pl.ds(base, RPT)], deg_hbm.at[cid, pl.ds(base, RPT)])


def kernel(x, edge_index, W_msg, b_msg):
    w1t = W_msg[:, :D].T
    w2t = W_msg[:, D:].T
    a, bp = _pre_call(x, w1t, w2t, b_msg.reshape(1, D))
    pad = EPWP - EPW
    row3 = jnp.pad(edge_index[0].reshape(NW, EPW), ((0, 0), (0, pad)),
                   constant_values=0).reshape(NW, NBLK, KB)
    col3 = jnp.pad(edge_index[1].reshape(NW, EPW), ((0, 0), (0, pad)),
                   constant_values=N).reshape(NW, NBLK, KB)
    partials, degp = _sc_scatter(a, row3, col3)
    return _post_call(partials, degp, bp, x)


# speedup vs baseline: 9.8890x; 1.0517x over previous
import functools

import jax
import jax.numpy as jnp
from jax import lax
from jax.experimental import pallas as pl
from jax.experimental.pallas import tpu as pltpu
from jax.experimental.pallas import tpu_sc as plsc

N = 10000
NPAD = 10240
D = 128
E = 320000
NC = 2
NS = 16
NW = NC * NS
EPW = E // NW
KB = 128
NBLK = 80
NSLOT = 6
EPWP = NBLK * KB
RPT = NPAD // NS
CHUNK = 128
NCHUNK = RPT // CHUNK
LANES = 16
RB = 1000
GRID = N // RB


def _post_body(p_ref, degp_ref, x_ref, w1t_ref, w2t_ref, b_ref, o_ref):
    s = p_ref[0].astype(jnp.float32) + p_ref[1].astype(jnp.float32)
    d = degp_ref[0] + degp_ref[1]
    dcol = d[:, 0:1]
    xb = x_ref[...]
    msg1 = jnp.dot(s, w1t_ref[...], preferred_element_type=jnp.float32)
    bp = (
        jnp.dot(xb, w2t_ref[...], preferred_element_type=jnp.float32)
        + b_ref[...]
    )
    o_ref[...] = msg1 + xb + dcol * bp


_post_call = pl.pallas_call(
    _post_body,
    grid=(GRID,),
    in_specs=[
        pl.BlockSpec((NC, RB, D), lambda i: (0, i, 0)),
        pl.BlockSpec((NC, RB, LANES), lambda i: (0, i, 0)),
        pl.BlockSpec((RB, D), lambda i: (i, 0)),
        pl.BlockSpec((D, D), lambda i: (0, 0)),
        pl.BlockSpec((D, D), lambda i: (0, 0)),
        pl.BlockSpec((1, D), lambda i: (0, 0)),
    ],
    out_specs=pl.BlockSpec((RB, D), lambda i: (i, 0)),
    out_shape=jax.ShapeDtypeStruct((N, D), jnp.float32),
)


_MESH = plsc.VectorSubcoreMesh(
    core_axis_name="c", subcore_axis_name="s", num_cores=NC, num_subcores=NS
)


def _fill_rows(ref, nrows, ncols, val, dtype):
    lanes = 32 if dtype == jnp.bfloat16 else LANES
    vec = jnp.full((lanes,), val, dtype)

    def body(i, carry):
        for j in range(ncols // lanes):
            ref[i, pl.ds(j * lanes, lanes)] = vec
        return carry

    lax.fori_loop(0, nrows, body, 0)


@functools.partial(
    pl.kernel,
    out_type=[
        jax.ShapeDtypeStruct((NC, NPAD, D), jnp.bfloat16),
        jax.ShapeDtypeStruct((NC, NPAD, LANES), jnp.float32),
    ],
    mesh=_MESH,
    compiler_params=pltpu.CompilerParams(use_tc_tiling_on_sc=False),
    scratch_types=[
        pltpu.VMEM((NBLK, KB), jnp.int32),
        pltpu.VMEM((NBLK, KB), jnp.int32),
        [pltpu.VMEM((KB, D), jnp.bfloat16)] * NSLOT,
        pltpu.VMEM((KB, LANES), jnp.float32),
        pltpu.VMEM_SHARED((NPAD, D), jnp.bfloat16),
        pltpu.VMEM_SHARED((NPAD, LANES), jnp.float32),
        [pltpu.SemaphoreType.DMA] * NSLOT,
        [pltpu.SemaphoreType.DMA] * NSLOT,
        [pltpu.SemaphoreType.DMA] * NSLOT,
    ],
)
def _sc_scatter(x_hbm, row_hbm, col_hbm, out_hbm, deg_hbm,
                row_v, col_v, gs, ones_v, acc, dacc,
                gsems, ssems, osems):
    cid = lax.axis_index("c")
    sid = lax.axis_index("s")
    wid = sid * NC + cid
    half = NSLOT // 2

    pltpu.sync_copy(row_hbm.at[wid], row_v)
    pltpu.sync_copy(col_hbm.at[wid], col_v)

    _fill_rows(gs[0], KB, D, 0.0, jnp.bfloat16)
    _fill_rows(ones_v, KB, LANES, 0.0, jnp.float32)
    for ch in range(NCHUNK):
        base = sid * RPT + ch * CHUNK
        pltpu.sync_copy(gs[0], acc.at[pl.ds(base, CHUNK)])
        pltpu.sync_copy(ones_v, dacc.at[pl.ds(base, CHUNK), :])
    _fill_rows(ones_v, KB, LANES, 1.0, jnp.float32)
    plsc.subcore_barrier()

    def wait_gather(b):
        pltpu.make_async_copy(x_hbm.at[pl.ds(0, KB)], gs[b], gsems[b]).wait()

    def wait_scatter(b):
        pltpu.make_async_copy(x_hbm.at[pl.ds(0, KB)], gs[b], ssems[b]).wait()

    def wait_ones(b):
        pltpu.make_async_copy(
            deg_hbm.at[cid, pl.ds(0, KB)], ones_v, osems[b]
        ).wait()

    def start_gather(b, j):
        pltpu.async_copy(x_hbm.at[row_v.at[j]], gs[b], gsems[b])

    def visit(b, j, first):
        wait_gather(b)

        @pl.when(jnp.logical_not(first))
        def _():
            wait_ones(b)

        pltpu.async_copy(gs[b], acc.at[col_v.at[j]], ssems[b], add=True)
        pltpu.async_copy(ones_v, dacc.at[col_v.at[j]], osems[b], add=True)

    for b in range(half):
        start_gather(b, b)

    def body(i, carry):
        first = i == 0
        j0 = NSLOT * i
        for b in range(half, NSLOT):
            start_gather(b, j0 + b)
        for b in range(half):
            visit(b, j0 + b, first)
        for b in range(half):
            wait_scatter(b)

        @pl.when(i < NBLK // NSLOT - 1)
        def _():
            for b in range(half):
                start_gather(b, j0 + NSLOT + b)

        for b in range(half, NSLOT):
            visit(b, j0 + b, first)
        for b in range(half, NSLOT):
            wait_scatter(b)
        return carry

    lax.fori_loop(0, NBLK // NSLOT, body, 0)

    tail = NBLK - (NBLK // NSLOT) * NSLOT
    for t in range(tail):
        b = t % NSLOT
        start_gather(b, NBLK - tail + t)
        visit(b, NBLK - tail + t, False)
        wait_scatter(b)

    for b in range(NSLOT):
        wait_ones(b)
    plsc.subcore_barrier()

    base = sid * RPT
    pltpu.sync_copy(acc.at[pl.ds(base, RPT)], out_hbm.at[cid, pl.ds(base, RPT)])
    pltpu.sync_copy(dacc.at[pl.ds(base, RPT)], deg_hbm.at[cid, pl.ds(base, RPT)])


def kernel(x, edge_index, W_msg, b_msg):
    w1t = W_msg[:, :D].T
    w2t = W_msg[:, D:].T
    xb = x.astype(jnp.bfloat16)
    pad = EPWP - EPW
    row3 = jnp.pad(edge_index[0].reshape(NW, EPW), ((0, 0), (0, pad)),
                   constant_values=0).reshape(NW, NBLK, KB)
    col3 = jnp.pad(edge_index[1].reshape(NW, EPW), ((0, 0), (0, pad)),
                   constant_values=N).reshape(NW, NBLK, KB)
    partials, degp = _sc_scatter(xb, row3, col3)
    return _post_call(partials, degp, x, w1t, w2t, b_msg.reshape(1, D))


# speedup vs baseline: 15.4118x; 1.5585x over previous
import functools

import jax
import jax.numpy as jnp
from jax import lax
from jax.experimental import pallas as pl
from jax.experimental.pallas import tpu as pltpu
from jax.experimental.pallas import tpu_sc as plsc

N = 10000
NPAD = 10240
D = 128
E = 320000
NC = 2
NS = 16
NW = NC * NS
EPW = E // NW
KB = 128
NBLK = 80
NHALF = NBLK // 2
EPWP = NBLK * KB
RPT = NPAD // NS
CHUNK = 128
NCHUNK = RPT // CHUNK
LANES = 16
RB = 1000
GRID = N // RB


def _post_body(p_ref, degp_ref, x_ref, w1t_ref, w2t_ref, b_ref, o_ref):
    s = p_ref[0].astype(jnp.float32) + p_ref[1].astype(jnp.float32)
    d = degp_ref[0] + degp_ref[1]
    dcol = d[:, 0:1]
    xb = x_ref[...]
    msg1 = jnp.dot(s, w1t_ref[...], preferred_element_type=jnp.float32)
    bp = (
        jnp.dot(xb, w2t_ref[...], preferred_element_type=jnp.float32)
        + b_ref[...]
    )
    o_ref[...] = msg1 + xb + dcol * bp


_post_call = pl.pallas_call(
    _post_body,
    grid=(GRID,),
    in_specs=[
        pl.BlockSpec((NC, RB, D), lambda i: (0, i, 0)),
        pl.BlockSpec((NC, RB, LANES), lambda i: (0, i, 0)),
        pl.BlockSpec((RB, D), lambda i: (i, 0)),
        pl.BlockSpec((D, D), lambda i: (0, 0)),
        pl.BlockSpec((D, D), lambda i: (0, 0)),
        pl.BlockSpec((1, D), lambda i: (0, 0)),
    ],
    out_specs=pl.BlockSpec((RB, D), lambda i: (i, 0)),
    out_shape=jax.ShapeDtypeStruct((N, D), jnp.float32),
)


_MESH = plsc.VectorSubcoreMesh(
    core_axis_name="c", subcore_axis_name="s", num_cores=NC, num_subcores=NS
)


def _fill_rows(ref, nrows, ncols, val, dtype):
    lanes = 32 if dtype == jnp.bfloat16 else LANES
    vec = jnp.full((lanes,), val, dtype)

    def body(i, carry):
        for j in range(ncols // lanes):
            ref[i, pl.ds(j * lanes, lanes)] = vec
        return carry

    lax.fori_loop(0, nrows, body, 0)


@functools.partial(
    pl.kernel,
    out_type=[
        jax.ShapeDtypeStruct((NC, NPAD, D), jnp.bfloat16),
        jax.ShapeDtypeStruct((NC, NPAD, LANES), jnp.float32),
    ],
    mesh=_MESH,
    compiler_params=pltpu.CompilerParams(use_tc_tiling_on_sc=False),
    scratch_types=[
        pltpu.VMEM((NHALF, KB), jnp.int32),
        pltpu.VMEM((NHALF, KB), jnp.int32),
        [pltpu.VMEM((KB, D), jnp.bfloat16)] * 2,
        pltpu.VMEM((KB, LANES), jnp.float32),
        pltpu.VMEM_SHARED((NPAD, D), jnp.bfloat16),
        pltpu.VMEM_SHARED((NPAD, D), jnp.bfloat16),
        pltpu.VMEM_SHARED((NPAD, LANES), jnp.float32),
        [pltpu.SemaphoreType.DMA] * 2,
        [pltpu.SemaphoreType.DMA] * 2,
        [pltpu.SemaphoreType.DMA] * 2,
    ],
)
def _sc_scatter(x_hbm, row_hbm, col_hbm, out_hbm, deg_hbm,
                row_v, col_v, gs, ones_v, xtab, acc, dacc,
                gsems, ssems, osems):
    cid = lax.axis_index("c")
    sid = lax.axis_index("s")
    wid = sid * NC + cid
    base = sid * RPT

    pltpu.sync_copy(x_hbm.at[pl.ds(base, RPT)], xtab.at[pl.ds(base, RPT)])
    _fill_rows(gs[0], KB, D, 0.0, jnp.bfloat16)
    _fill_rows(ones_v, KB, LANES, 0.0, jnp.float32)
    for ch in range(NCHUNK):
        cb = base + ch * CHUNK
        pltpu.sync_copy(gs[0], acc.at[pl.ds(cb, CHUNK)])
        pltpu.sync_copy(ones_v, dacc.at[pl.ds(cb, CHUNK), :])
    _fill_rows(ones_v, KB, LANES, 1.0, jnp.float32)
    plsc.subcore_barrier()

    def wait_gather(b):
        pltpu.make_async_copy(x_hbm.at[pl.ds(0, KB)], gs[b], gsems[b]).wait()

    def wait_scatter(b):
        pltpu.make_async_copy(x_hbm.at[pl.ds(0, KB)], gs[b], ssems[b]).wait()

    def wait_ones(b):
        pltpu.make_async_copy(
            deg_hbm.at[cid, pl.ds(0, KB)], ones_v, osems[b]
        ).wait()

    def start_gather(b, j):
        pltpu.async_copy(xtab.at[row_v.at[j]], gs[b], gsems[b])

    def visit(b, j, first):
        wait_gather(b)

        @pl.when(jnp.logical_not(first))
        def _():
            wait_ones(b)

        pltpu.async_copy(gs[b], acc.at[col_v.at[j]], ssems[b], add=True)
        pltpu.async_copy(ones_v, dacc.at[col_v.at[j]], osems[b], add=True)

    for p in range(2):
        pltpu.sync_copy(row_hbm.at[wid, pl.ds(p * NHALF, NHALF)], row_v)
        pltpu.sync_copy(col_hbm.at[wid, pl.ds(p * NHALF, NHALF)], col_v)
        start_gather(0, 0)

        def body(i, carry, p=p):
            first = i == 0
            start_gather(1, 2 * i + 1)
            visit(0, 2 * i, first)
            wait_scatter(0)

            @pl.when(i < NHALF // 2 - 1)
            def _():
                start_gather(0, 2 * i + 2)

            visit(1, 2 * i + 1, first)
            wait_scatter(1)
            return carry

        lax.fori_loop(0, NHALF // 2, body, 0)
        for b in range(2):
            wait_ones(b)

    plsc.subcore_barrier()

    pltpu.sync_copy(acc.at[pl.ds(base, RPT)], out_hbm.at[cid, pl.ds(base, RPT)])
    pltpu.sync_copy(dacc.at[pl.ds(base, RPT)], deg_hbm.at[cid, pl.ds(base, RPT)])


def kernel(x, edge_index, W_msg, b_msg):
    w1t = W_msg[:, :D].T
    w2t = W_msg[:, D:].T
    xb = jnp.pad(x.astype(jnp.bfloat16), ((0, NPAD - N), (0, 0)))
    pad = EPWP - EPW
    row3 = jnp.pad(edge_index[0].reshape(NW, EPW), ((0, 0), (0, pad)),
                   constant_values=0).reshape(NW, NBLK, KB)
    col3 = jnp.pad(edge_index[1].reshape(NW, EPW), ((0, 0), (0, pad)),
                   constant_values=N).reshape(NW, NBLK, KB)
    partials, degp = _sc_scatter(xb, row3, col3)
    return _post_call(partials, degp, x, w1t, w2t, b_msg.reshape(1, D))


# speedup vs baseline: 17.2749x; 1.1209x over previous
import functools

import jax
import jax.numpy as jnp
from jax import lax
from jax.experimental import pallas as pl
from jax.experimental.pallas import tpu as pltpu
from jax.experimental.pallas import tpu_sc as plsc

N = 10000
NPAD = 10240
D = 128
E = 320000
NC = 2
NS = 16
NW = NC * NS
EPW = E // NW
KB = 80
NBLK = EPW // KB
NSLOT = 3
RPT = NPAD // NS
XR0 = (NS - 1) * (NPAD // NS)
CHUNK = 128
NCHUNK = RPT // CHUNK
LANES = 16
RB = 1000
GRID = N // RB


def _post_body(p_ref, degp_ref, x_ref, w1_ref, w2_ref, b_ref, o_ref):
    s = p_ref[0].astype(jnp.float32) + p_ref[1].astype(jnp.float32)
    d = degp_ref[0] + degp_ref[1]
    dcol = d[:, 0:1]
    xb = x_ref[...]
    dn = (((1,), (1,)), ((), ()))
    msg1 = lax.dot_general(s, w1_ref[...], dn,
                           preferred_element_type=jnp.float32)
    bp = (
        lax.dot_general(xb, w2_ref[...], dn,
                        preferred_element_type=jnp.float32)
        + b_ref[...]
    )
    o_ref[...] = msg1 + xb + dcol * bp


_post_call = pl.pallas_call(
    _post_body,
    grid=(GRID,),
    in_specs=[
        pl.BlockSpec((NC, RB, D), lambda i: (0, i, 0)),
        pl.BlockSpec((NC, RB, LANES), lambda i: (0, i, 0)),
        pl.BlockSpec((RB, D), lambda i: (i, 0)),
        pl.BlockSpec((D, D), lambda i: (0, 0)),
        pl.BlockSpec((D, D), lambda i: (0, 1)),
        pl.BlockSpec((1, D), lambda i: (0, 0)),
    ],
    out_specs=pl.BlockSpec((RB, D), lambda i: (i, 0)),
    out_shape=jax.ShapeDtypeStruct((N, D), jnp.float32),
)


_MESH = plsc.VectorSubcoreMesh(
    core_axis_name="c", subcore_axis_name="s", num_cores=NC, num_subcores=NS
)


def _fill_rows(ref, nrows, ncols, val, dtype):
    lanes = 32 if dtype == jnp.bfloat16 else LANES
    vec = jnp.full((lanes,), val, dtype)

    def body(i, carry):
        for j in range(ncols // lanes):
            ref[i, pl.ds(j * lanes, lanes)] = vec
        return carry

    lax.fori_loop(0, nrows, body, 0)


@functools.partial(
    pl.kernel,
    out_type=[
        jax.ShapeDtypeStruct((NC, NPAD, D), jnp.bfloat16),
        jax.ShapeDtypeStruct((NC, NPAD, LANES), jnp.float32),
    ],
    mesh=_MESH,
    compiler_params=pltpu.CompilerParams(use_tc_tiling_on_sc=False),
    scratch_types=[
        pltpu.VMEM((NBLK, KB), jnp.int32),
        pltpu.VMEM((NBLK, KB), jnp.int32),
        [pltpu.VMEM((KB, D), jnp.bfloat16)] * NSLOT,
        pltpu.VMEM((KB, LANES), jnp.float32),
        pltpu.VMEM_SHARED((NPAD, D), jnp.bfloat16),
        pltpu.VMEM_SHARED((NPAD, D), jnp.bfloat16),
        pltpu.VMEM_SHARED((NPAD, LANES), jnp.float32),
        [pltpu.SemaphoreType.DMA] * NSLOT,
        [pltpu.SemaphoreType.DMA] * NSLOT,
        [pltpu.SemaphoreType.DMA] * NSLOT,
    ],
)
def _sc_scatter(x_hbm, row_hbm, col_hbm, out_hbm, deg_hbm,
                row_v, col_v, gs, ones_v, xtab, acc, dacc,
                gsems, ssems, osems):
    cid = lax.axis_index("c")
    sid = lax.axis_index("s")
    wid = sid * NC + cid
    base = sid * RPT

    pltpu.sync_copy(row_hbm.at[wid], row_v)
    pltpu.sync_copy(col_hbm.at[wid], col_v)

    @pl.when(sid < NS - 1)
    def _():
        pltpu.sync_copy(x_hbm.at[pl.ds(base, RPT)], xtab.at[pl.ds(base, RPT)])

    @pl.when(sid == NS - 1)
    def _():
        pltpu.sync_copy(x_hbm.at[pl.ds(XR0, N - XR0)],
                        xtab.at[pl.ds(XR0, N - XR0)])

    _fill_rows(gs[0], KB, D, 0.0, jnp.bfloat16)
    _fill_rows(ones_v, KB, LANES, 0.0, jnp.float32)
    for ch in range(RPT // KB):
        cb = base + ch * KB
        pltpu.sync_copy(gs[0], acc.at[pl.ds(cb, KB)])
        pltpu.sync_copy(ones_v, dacc.at[pl.ds(cb, KB), :])
    _fill_rows(ones_v, KB, LANES, 1.0, jnp.float32)
    plsc.subcore_barrier()

    def wait_gather(b):
        pltpu.make_async_copy(x_hbm.at[pl.ds(0, KB)], gs[b], gsems[b]).wait()

    def wait_scatter(b):
        pltpu.make_async_copy(x_hbm.at[pl.ds(0, KB)], gs[b], ssems[b]).wait()

    def wait_ones(b):
        pltpu.make_async_copy(
            deg_hbm.at[cid, pl.ds(0, KB)], ones_v, osems[b]
        ).wait()

    def start_gather(b, j):
        pltpu.async_copy(xtab.at[row_v.at[j]], gs[b], gsems[b])

    def visit(b, j):
        wait_gather(b)

        @pl.when(j >= NSLOT)
        def _():
            wait_ones(b)

        pltpu.async_copy(gs[b], acc.at[col_v.at[j]], ssems[b], add=True)
        pltpu.async_copy(ones_v, dacc.at[col_v.at[j]], osems[b], add=True)

    for b in range(NSLOT):
        start_gather(b, b)

    def body(i, carry):
        for b in range(NSLOT):
            j = NSLOT * i + b
            visit(b, j)
            wait_scatter(b)

            @pl.when(j + NSLOT < NBLK)
            def _():
                start_gather(b, j + NSLOT)
        return carry

    lax.fori_loop(0, NBLK // NSLOT, body, 0)
    for t in range(NBLK - (NBLK // NSLOT) * NSLOT):
        visit(t, (NBLK // NSLOT) * NSLOT + t)
        wait_scatter(t)

    for b in range(NSLOT):
        wait_ones(b)
    plsc.subcore_barrier()

    pltpu.sync_copy(acc.at[pl.ds(base, RPT)], out_hbm.at[cid, pl.ds(base, RPT)])
    pltpu.sync_copy(dacc.at[pl.ds(base, RPT)], deg_hbm.at[cid, pl.ds(base, RPT)])


def kernel(x, edge_index, W_msg, b_msg):
    xb = x.astype(jnp.bfloat16)
    row3 = edge_index[0].reshape(NW, NBLK, KB)
    col3 = edge_index[1].reshape(NW, NBLK, KB)
    partials, degp = _sc_scatter(xb, row3, col3)
    return _post_call(partials, degp, x, W_msg, W_msg, b_msg.reshape(1, D))
